# baseline (speedup 1.0000x reference)
import functools

import jax
import jax.numpy as jnp
from jax.experimental import pallas as pl
from jax.experimental.pallas import tpu as pltpu


def _moe_body(idx_ref, wv_ref, x_hbm, wr_ref, ws_ref, bs_ref, br_ref,
              out_hbm, x_buf, o_buf, in_sems, out_sems, *, nb, nbuf, nsteps):
    i = pl.program_id(0)
    slot = jax.lax.rem(i, nbuf)
    la = nbuf - 1

    def start_in(step):
        s = jax.lax.rem(step, nbuf)
        for j in range(nb):
            pltpu.make_async_copy(
                x_hbm.at[step * nb + j], x_buf.at[s, j], in_sems.at[s, j]
            ).start()

    @pl.when(i == 0)
    def _():
        for k in range(la):
            start_in(k)

    @pl.when(i + la < nsteps)
    def _():
        start_in(i + la)

    @pl.when(i >= nbuf)
    def _():
        for j in range(nb):
            pltpu.make_async_copy(
                o_buf.at[slot, j], out_hbm.at[(i - nbuf) * nb + j],
                out_sems.at[slot, j],
            ).wait()

    for j in range(nb):
        pltpu.make_async_copy(
            x_hbm.at[i * nb + j], x_buf.at[slot, j], in_sems.at[slot, j]
        ).wait()

    for j in range(nb):
        e = idx_ref[i * nb + j]
        w = wv_ref[i * nb + j]
        wc = ws_ref[...] + w.astype(jnp.bfloat16) * wr_ref[e]
        acc = jnp.dot(wc, x_buf[slot, j].astype(jnp.bfloat16),
                      preferred_element_type=jnp.float32)
        o_buf[slot, j] = acc + (bs_ref[...] + w * br_ref[e])
        pltpu.make_async_copy(
            o_buf.at[slot, j], out_hbm.at[i * nb + j], out_sems.at[slot, j]
        ).start()

    @pl.when(i == nsteps - 1)
    def _():
        for s in range(nbuf):
            for j in range(nb):
                pltpu.make_async_copy(
                    o_buf.at[s, j], out_hbm.at[j], out_sems.at[s, j]
                ).wait()


def kernel(x, weights, indices, Ws, bs, Wr, br):
    B, C, H, W = x.shape
    E, O, _ = Wr.shape
    HW = H * W
    nb, nbuf = 4, 3
    nsteps = B // nb

    xf = x.reshape(B, C, HW)
    idx = indices.reshape(-1).astype(jnp.int32)
    wv = weights.reshape(-1).astype(jnp.float32)
    wr16 = Wr.astype(jnp.bfloat16)
    ws16 = Ws.astype(jnp.bfloat16)
    bs2 = bs.reshape(O, 1)
    br2 = br.reshape(E, O, 1)

    grid_spec = pltpu.PrefetchScalarGridSpec(
        num_scalar_prefetch=2,
        grid=(nsteps,),
        in_specs=[
            pl.BlockSpec(memory_space=pl.ANY),
            pl.BlockSpec((E, O, C), lambda b, i, w: (0, 0, 0)),
            pl.BlockSpec((O, C), lambda b, i, w: (0, 0)),
            pl.BlockSpec((O, 1), lambda b, i, w: (0, 0)),
            pl.BlockSpec((E, O, 1), lambda b, i, w: (0, 0, 0)),
        ],
        out_specs=pl.BlockSpec(memory_space=pl.ANY),
        scratch_shapes=[
            pltpu.VMEM((nbuf, nb, C, HW), jnp.float32),
            pltpu.VMEM((nbuf, nb, O, HW), jnp.float32),
            pltpu.SemaphoreType.DMA((nbuf, nb)),
            pltpu.SemaphoreType.DMA((nbuf, nb)),
        ],
    )
    out = pl.pallas_call(
        functools.partial(_moe_body, nb=nb, nbuf=nbuf, nsteps=nsteps),
        grid_spec=grid_spec,
        out_shape=jax.ShapeDtypeStruct((B, O, HW), jnp.float32),
    )(idx, wv, xf, wr16, ws16, bs2, br2)
    return out.reshape(B, O, H, W)
